# baseline (speedup 1.0000x reference)
import functools

import jax
import jax.numpy as jnp
from jax import lax
from jax.experimental import pallas as pl
from jax.experimental.pallas import tpu as pltpu
from jax.experimental.pallas import tpu_sc as plsc

_N_WORKERS = 32
_CHUNK_ROWS = 16
_N_BUF = 6


@functools.lru_cache(maxsize=None)
def _make_copy(num_rows: int, dim: int):
    rows_per_w = num_rows // _N_WORKERS
    n_chunks = rows_per_w // _CHUNK_ROWS
    mesh = plsc.VectorSubcoreMesh(core_axis_name="c", subcore_axis_name="s")

    @functools.partial(
        pl.kernel,
        out_type=jax.ShapeDtypeStruct((num_rows, dim), jnp.float32),
        mesh=mesh,
        scratch_types=(
            [pltpu.VMEM((_CHUNK_ROWS, dim), jnp.float32) for _ in range(_N_BUF)]
            + [pltpu.SemaphoreType.DMA for _ in range(2 * _N_BUF)]
        ),
    )
    def copy_kernel(table_hbm, out_hbm, *scratch):
        bufs = scratch[:_N_BUF]
        in_sems = scratch[_N_BUF:2 * _N_BUF]
        out_sems = scratch[2 * _N_BUF:]
        wid = lax.axis_index("s") * 2 + lax.axis_index("c")
        base = wid * rows_per_w

        def src(c):
            return table_hbm.at[pl.ds(base + c * _CHUNK_ROWS, _CHUNK_ROWS)]

        def dst(c):
            return out_hbm.at[pl.ds(base + c * _CHUNK_ROWS, _CHUNK_ROWS)]

        for b in range(min(_N_BUF, n_chunks)):
            pltpu.async_copy(src(b), bufs[b], in_sems[b])
        for c in range(n_chunks):
            b = c % _N_BUF
            pltpu.make_async_copy(src(c), bufs[b], in_sems[b]).wait()
            pltpu.async_copy(bufs[b], dst(c), out_sems[b])
            nxt = c + 1
            if _N_BUF <= nxt < n_chunks:
                prev = nxt - _N_BUF
                pb = prev % _N_BUF
                pltpu.make_async_copy(bufs[pb], dst(prev), out_sems[pb]).wait()
                pltpu.async_copy(src(nxt), bufs[pb], in_sems[pb])
        for c in range(max(0, n_chunks - _N_BUF), n_chunks):
            b = c % _N_BUF
            pltpu.make_async_copy(bufs[b], dst(c), out_sems[b]).wait()

    return copy_kernel


def kernel(x, table):
    num_rows, dim = table.shape
    out = _make_copy(num_rows, dim)(table)
    return out[None]


# speedup vs baseline: 1.0669x; 1.0669x over previous
import functools

import jax
import jax.numpy as jnp
from jax import lax
from jax.experimental import pallas as pl
from jax.experimental.pallas import tpu as pltpu
from jax.experimental.pallas import tpu_sc as plsc

_N_WORKERS = 32
_N_BUF = 2
_SPMEM_WORDS = 131071


def _chunk_sizes(rows_per_w: int, dim: int, n_buf: int):
    max_rows = (_SPMEM_WORDS // (dim * n_buf)) & ~7
    n_chunks = -(-rows_per_w // max_rows)
    units, rem = divmod(rows_per_w // 8, n_chunks)
    return [8 * (units + (1 if i < rem else 0)) for i in range(n_chunks)]


@functools.lru_cache(maxsize=None)
def _make_copy(num_rows: int, dim: int):
    rows_per_w = num_rows // _N_WORKERS
    sizes = _chunk_sizes(rows_per_w, dim, _N_BUF)
    n_chunks = len(sizes)
    offs = [sum(sizes[:i]) for i in range(n_chunks)]
    buf_rows = max(sizes)
    mesh = plsc.VectorSubcoreMesh(core_axis_name="c", subcore_axis_name="s")

    @functools.partial(
        pl.kernel,
        out_type=jax.ShapeDtypeStruct((num_rows, dim), jnp.float32),
        mesh=mesh,
        scratch_types=(
            [pltpu.VMEM((buf_rows, dim), jnp.float32) for _ in range(_N_BUF)]
            + [pltpu.SemaphoreType.DMA for _ in range(2 * _N_BUF)]
        ),
    )
    def copy_kernel(table_hbm, out_hbm, *scratch):
        bufs = scratch[:_N_BUF]
        in_sems = scratch[_N_BUF:2 * _N_BUF]
        out_sems = scratch[2 * _N_BUF:]
        wid = lax.axis_index("s") * 2 + lax.axis_index("c")
        base = wid * rows_per_w

        def src(c):
            return table_hbm.at[pl.ds(base + offs[c], sizes[c])]

        def dst(c):
            return out_hbm.at[pl.ds(base + offs[c], sizes[c])]

        def buf(c):
            return bufs[c % _N_BUF].at[pl.ds(0, sizes[c])]

        for b in range(min(_N_BUF, n_chunks)):
            pltpu.async_copy(src(b), buf(b), in_sems[b])
        for c in range(n_chunks):
            b = c % _N_BUF
            pltpu.make_async_copy(src(c), buf(c), in_sems[b]).wait()
            pltpu.async_copy(buf(c), dst(c), out_sems[b])
            nxt = c + 1
            if _N_BUF <= nxt < n_chunks:
                prev = nxt - _N_BUF
                pb = prev % _N_BUF
                pltpu.make_async_copy(buf(prev), dst(prev), out_sems[pb]).wait()
                pltpu.async_copy(src(nxt), buf(nxt), in_sems[pb])
        for c in range(max(0, n_chunks - _N_BUF), n_chunks):
            b = c % _N_BUF
            pltpu.make_async_copy(buf(c), dst(c), out_sems[b]).wait()

    return copy_kernel


def kernel(x, table):
    num_rows, dim = table.shape
    out = _make_copy(num_rows, dim)(table)
    return out[None]


# speedup vs baseline: 1.8764x; 1.7587x over previous
import functools
import jax
import jax.numpy as jnp
from jax.experimental import pallas as pl


def _copy_body(t_ref, o_ref):
    o_ref[...] = t_ref[...]


@functools.lru_cache(maxsize=None)
def _make_copy(num_rows: int, dim: int, block: int):
    return pl.pallas_call(
        _copy_body,
        grid=(num_rows // block,),
        in_specs=[pl.BlockSpec((block, dim), lambda i: (i, 0))],
        out_specs=pl.BlockSpec((block, dim), lambda i: (i, 0)),
        out_shape=jax.ShapeDtypeStruct((num_rows, dim), jnp.float32),
    )


def kernel(x, table):
    num_rows, dim = table.shape
    out = _make_copy(num_rows, dim, 512)(table)
    return out[None]
